# Initial kernel scaffold; baseline (speedup 1.0000x reference)
import jax
import jax.numpy as jnp
from jax.experimental import pallas as pl


def kernel(z_t, t, x_0, x_1):
    raise NotImplementedError("write your pallas kernel here")


# speedup vs baseline: 324.9398x; 324.9398x over previous
import functools

import jax
import jax.numpy as jnp
from jax.experimental import pallas as pl
from jax.experimental.pallas import tpu as pltpu

_B = 1024
_M = 2048
_D = 64
_H = 1.0
_BLK_B = 256


def _velocity_kernel(t_ref, z_ref, x0_ref, x1_ref, out_ref):
    tv = t_ref[0]
    zb = z_ref[...]
    x0 = x0_ref[...]
    x1 = x1_ref[...]

    x_t = (1.0 - tv) * x0 + tv * x1

    g = jax.lax.dot_general(
        zb, x_t, (((1,), (1,)), ((), ())),
        precision=jax.lax.Precision.HIGHEST,
        preferred_element_type=jnp.float32)
    zn2 = jnp.sum(zb * zb, axis=1, keepdims=True)
    xn2 = jnp.sum(x_t * x_t, axis=1)[None, :]
    dsq = zn2 + xn2 - 2.0 * g

    inv_2h2 = 1.0 / (2.0 * _H * _H)
    s = jnp.exp(-dsq * inv_2h2)
    sum_s = jnp.sum(s, axis=1, keepdims=True)

    num = jax.lax.dot_general(
        s, x1, (((1,), (0,)), ((), ())),
        precision=jax.lax.Precision.HIGHEST,
        preferred_element_type=jnp.float32)

    scale = 1.0 / ((sum_s + 1e-7) * (1.0 - tv + 1e-7))
    out_ref[...] = (num - zb * sum_s) * scale


@jax.jit
def kernel(z_t, t, x_0, x_1):
    tv = t.reshape(-1)[:1]
    grid = _B // _BLK_B
    return pl.pallas_call(
        _velocity_kernel,
        grid=(grid,),
        in_specs=[
            pl.BlockSpec(memory_space=pltpu.SMEM),
            pl.BlockSpec((_BLK_B, _D), lambda i: (i, 0)),
            pl.BlockSpec((_M, _D), lambda i: (0, 0)),
            pl.BlockSpec((_M, _D), lambda i: (0, 0)),
        ],
        out_specs=pl.BlockSpec((_BLK_B, _D), lambda i: (i, 0)),
        out_shape=jax.ShapeDtypeStruct((_B, _D), jnp.float32),
    )(tv, z_t, x_0, x_1)


# speedup vs baseline: 429.8294x; 1.3228x over previous
import functools

import jax
import jax.numpy as jnp
from jax.experimental import pallas as pl
from jax.experimental.pallas import tpu as pltpu

_B = 1024
_M = 2048
_D = 64
_H = 1.0
_BLK_B = 256


def _split_bf16(a):
    hi = a.astype(jnp.bfloat16)
    lo = (a - hi.astype(jnp.float32)).astype(jnp.bfloat16)
    return hi, lo


def _dot3(a, b, dims):
    a_hi, a_lo = _split_bf16(a)
    b_hi, b_lo = _split_bf16(b)
    dn = (dims, ((), ()))
    acc = jax.lax.dot_general(
        a_hi, b_hi, dn, preferred_element_type=jnp.float32)
    acc += jax.lax.dot_general(
        a_hi, b_lo, dn, preferred_element_type=jnp.float32)
    acc += jax.lax.dot_general(
        a_lo, b_hi, dn, preferred_element_type=jnp.float32)
    return acc


def _velocity_kernel(t_ref, z_ref, x0_ref, x1a_ref, out_ref):
    tv = t_ref[0]
    zb = z_ref[...]
    x0 = x0_ref[...]
    x1a = x1a_ref[...]
    x1 = x1a[:, :_D]

    x_t = (1.0 - tv) * x0 + tv * x1

    g = _dot3(zb, x_t, ((1,), (1,)))
    zn2 = jnp.sum(zb * zb, axis=1, keepdims=True)
    xn2 = jnp.sum(x_t * x_t, axis=1)[None, :]
    dsq = zn2 + xn2 - 2.0 * g

    inv_2h2 = 1.0 / (2.0 * _H * _H)
    s = jnp.exp(-dsq * inv_2h2)

    num_aug = _dot3(s, x1a, ((1,), (0,)))
    num = num_aug[:, :_D]
    sum_s = num_aug[:, _D:]

    scale = 1.0 / ((sum_s + 1e-7) * (1.0 - tv + 1e-7))
    out_ref[...] = (num - zb * sum_s) * scale


@jax.jit
def kernel(z_t, t, x_0, x_1):
    tv = t.reshape(-1)[:1]
    x1a = jnp.concatenate(
        [x_1, jnp.ones((_M, 1), jnp.float32)], axis=1)
    grid = _B // _BLK_B
    return pl.pallas_call(
        _velocity_kernel,
        grid=(grid,),
        in_specs=[
            pl.BlockSpec(memory_space=pltpu.SMEM),
            pl.BlockSpec((_BLK_B, _D), lambda i: (i, 0)),
            pl.BlockSpec((_M, _D), lambda i: (0, 0)),
            pl.BlockSpec((_M, _D + 1), lambda i: (0, 0)),
        ],
        out_specs=pl.BlockSpec((_BLK_B, _D), lambda i: (i, 0)),
        out_shape=jax.ShapeDtypeStruct((_B, _D), jnp.float32),
    )(tv, z_t, x_0, x1a)
